# baseline (speedup 1.0000x reference)
import functools

import jax
import jax.numpy as jnp
from jax import lax
from jax.experimental import pallas as pl
from jax.experimental.pallas import tpu as pltpu
from jax.experimental.pallas import tpu_sc as plsc

_NC = 2
_NS = 16
_NW = _NC * _NS


@functools.lru_cache(maxsize=None)
def _make_gather(n, v, d, chunk):
    assert n % _NW == 0
    per_w = n // _NW
    assert per_w % chunk == 0
    mesh = plsc.VectorSubcoreMesh(core_axis_name="c", subcore_axis_name="s")

    @functools.partial(
        pl.kernel,
        mesh=mesh,
        out_type=jax.ShapeDtypeStruct((n, d), jnp.float32),
        scratch_types=[
            pltpu.VMEM((chunk,), jnp.int32),
            pltpu.VMEM((chunk, d), jnp.float32),
            pltpu.SemaphoreType.DMA,
        ],
        compiler_params=pltpu.CompilerParams(use_tc_tiling_on_sc=False),
    )
    def gather(table_hbm, idx_hbm, out_hbm, idx_v, rows_v, sem):
        wid = lax.axis_index("s") * _NC + lax.axis_index("c")
        base = wid * per_w

        def body(i, carry):
            off = base + i * chunk
            pltpu.sync_copy(idx_hbm.at[pl.ds(off, chunk)], idx_v)
            pltpu.async_copy(table_hbm.at[idx_v], rows_v, sem).wait()
            pltpu.sync_copy(rows_v, out_hbm.at[pl.ds(off, chunk)])
            return carry

        lax.fori_loop(0, per_w // chunk, body, 0)

    return gather


def kernel(input_, weight):
    b, h = input_.shape
    v, d = weight.shape
    n = b * h
    idx = input_.reshape(n)
    out = _make_gather(n, v, d, 1024)(weight, idx)
    return out.reshape(b, h, d)


# speedup vs baseline: 1.0032x; 1.0032x over previous
import functools

import jax
import jax.numpy as jnp
from jax import lax
from jax.experimental import pallas as pl
from jax.experimental.pallas import tpu as pltpu
from jax.experimental.pallas import tpu_sc as plsc

_NC = 2
_NS = 16
_NW = _NC * _NS


@functools.lru_cache(maxsize=None)
def _make_gather(n, v, d, chunk, nbuf):
    assert n % _NW == 0
    per_w = n // _NW
    assert per_w % (chunk * nbuf) == 0
    n_iters = per_w // (chunk * nbuf)
    mesh = plsc.VectorSubcoreMesh(core_axis_name="c", subcore_axis_name="s")

    @functools.partial(
        pl.kernel,
        mesh=mesh,
        out_type=jax.ShapeDtypeStruct((n, d), jnp.float32),
        scratch_types=[
            pltpu.VMEM((per_w,), jnp.int32),
            pltpu.VMEM((nbuf, chunk, d), jnp.float32),
            [pltpu.SemaphoreType.DMA] * nbuf,
            [pltpu.SemaphoreType.DMA] * nbuf,
        ],
        compiler_params=pltpu.CompilerParams(use_tc_tiling_on_sc=False),
    )
    def gather(table_hbm, idx_hbm, out_hbm, idx_v, rows_v, gsems, wsems):
        wid = lax.axis_index("s") * _NC + lax.axis_index("c")
        base = wid * per_w
        pltpu.sync_copy(idx_hbm.at[pl.ds(base, per_w)], idx_v)

        def body(i, carry):
            g0 = i * (chunk * nbuf)
            gh, wh = [], []
            for b in range(nbuf):
                off = g0 + b * chunk
                gh.append(
                    pltpu.async_copy(
                        table_hbm.at[idx_v.at[pl.ds(off, chunk)]],
                        rows_v.at[b],
                        gsems[b],
                    )
                )
            for b in range(nbuf):
                off = g0 + b * chunk
                gh[b].wait()
                wh.append(
                    pltpu.async_copy(
                        rows_v.at[b],
                        out_hbm.at[pl.ds(base + off, chunk)],
                        wsems[b],
                    )
                )
            for b in range(nbuf):
                wh[b].wait()
            return carry

        lax.fori_loop(0, n_iters, body, 0)

    return gather


def kernel(input_, weight):
    b, h = input_.shape
    v, d = weight.shape
    n = b * h
    idx = input_.reshape(n)
    out = _make_gather(n, v, d, 320, 4)(weight, idx)
    return out.reshape(b, h, d)
